# baseline (speedup 1.0000x reference)
import functools

import jax
import jax.numpy as jnp
from jax import lax
from jax.experimental import pallas as pl
from jax.experimental.pallas import tpu as pltpu
from jax.experimental.pallas import tpu_sc as plsc

_VOCAB = 100000
_EMBED = 64
_HIDDEN = 128
_BATCH = 4096
_SEQ = 50

_NC = 2
_NS = 16
_NW = _NC * _NS
_ROWS_PER_W = _BATCH // _NW
_CHUNK_ROWS = 2
_CHUNK_IDX = _CHUNK_ROWS * _SEQ
_NCHUNK = _ROWS_PER_W // _CHUNK_ROWS


def _pool_body(src_r, table, out, idx_v, rows_v, acc_v, sem):
    wid = lax.axis_index("s") * _NC + lax.axis_index("c")
    pltpu.sync_copy(src_r.at[wid], idx_v)

    def chunk(c, carry):
        pltpu.async_copy(table.at[idx_v.at[c]], rows_v, sem).wait()
        for r in range(_CHUNK_ROWS):
            for k in range(_EMBED // 16):
                acc = rows_v[r * _SEQ, pl.ds(16 * k, 16)]
                for s in range(1, _SEQ):
                    acc = acc + rows_v[r * _SEQ + s, pl.ds(16 * k, 16)]
                acc_v[_CHUNK_ROWS * c + r, pl.ds(16 * k, 16)] = acc * (1.0 / _SEQ)
        return carry

    lax.fori_loop(0, _NCHUNK, chunk, 0)
    pltpu.sync_copy(acc_v, out.at[pl.ds(wid * _ROWS_PER_W, _ROWS_PER_W)])


_pool_call = functools.partial(
    pl.kernel,
    mesh=plsc.VectorSubcoreMesh(core_axis_name="c", subcore_axis_name="s"),
    out_type=jax.ShapeDtypeStruct((_BATCH, _EMBED), jnp.float32),
    scratch_types=[
        pltpu.VMEM((_NCHUNK, _CHUNK_IDX), jnp.int32),
        pltpu.VMEM((_CHUNK_IDX, _EMBED), jnp.float32),
        pltpu.VMEM((_ROWS_PER_W, _EMBED), jnp.float32),
        pltpu.SemaphoreType.DMA,
    ],
    compiler_params=pltpu.CompilerParams(use_tc_tiling_on_sc=False),
)(_pool_body)


_BN = 1024


def _mlp_body(pooled_ref, W1_ref, b1_ref, W2_ref, b2_ref, out_ref, h_ref):
    @pl.when(pl.program_id(0) == 0)
    def _():
        h = jnp.dot(pooled_ref[...], W1_ref[...], preferred_element_type=jnp.float32)
        h_ref[...] = jnp.maximum(h + b1_ref[...], 0.0)

    out_ref[...] = (
        jnp.dot(h_ref[...], W2_ref[...], preferred_element_type=jnp.float32)
        + b2_ref[...]
    )


_mlp_call = pl.pallas_call(
    _mlp_body,
    grid=(pl.cdiv(_VOCAB, _BN),),
    in_specs=[
        pl.BlockSpec((_BATCH, _EMBED), lambda j: (0, 0)),
        pl.BlockSpec((_EMBED, _HIDDEN), lambda j: (0, 0)),
        pl.BlockSpec((1, _HIDDEN), lambda j: (0, 0)),
        pl.BlockSpec((_HIDDEN, _BN), lambda j: (0, j)),
        pl.BlockSpec((1, _BN), lambda j: (0, j)),
    ],
    out_specs=pl.BlockSpec((_BATCH, _BN), lambda j: (0, j)),
    out_shape=jax.ShapeDtypeStruct((_BATCH, _VOCAB), jnp.float32),
    scratch_shapes=[pltpu.VMEM((_BATCH, _HIDDEN), jnp.float32)],
)


def kernel(src, emb_table, W1, b1, W2, b2):
    src_r = src.reshape(_NW, _NCHUNK, _CHUNK_IDX).astype(jnp.int32)
    pooled = _pool_call(src_r, emb_table)
    return _mlp_call(pooled, W1, b1.reshape(1, -1), W2, b2.reshape(1, -1))


# speedup vs baseline: 1.0008x; 1.0008x over previous
import functools

import jax
import jax.numpy as jnp
from jax import lax
from jax.experimental import pallas as pl
from jax.experimental.pallas import tpu as pltpu
from jax.experimental.pallas import tpu_sc as plsc

_VOCAB = 100000
_EMBED = 64
_HIDDEN = 128
_BATCH = 4096
_SEQ = 50

_NC = 2
_NS = 16
_NW = _NC * _NS
_ROWS_PER_W = _BATCH // _NW
_CHUNK_ROWS = 2
_CHUNK_IDX = _CHUNK_ROWS * _SEQ
_NCHUNK = _ROWS_PER_W // _CHUNK_ROWS


def _pool_body(src_r, table, out, idx_v, rows_v, acc_v, sem):
    wid = lax.axis_index("s") * _NC + lax.axis_index("c")
    pltpu.sync_copy(src_r.at[wid], idx_v)

    def chunk(c, carry):
        pltpu.async_copy(table.at[idx_v.at[c]], rows_v, sem).wait()
        for r in range(_CHUNK_ROWS):
            for k in range(_EMBED // 16):
                acc = rows_v[r * _SEQ, pl.ds(16 * k, 16)]
                for s in range(1, _SEQ):
                    acc = acc + rows_v[r * _SEQ + s, pl.ds(16 * k, 16)]
                acc_v[_CHUNK_ROWS * c + r, pl.ds(16 * k, 16)] = acc * (1.0 / _SEQ)
        return carry

    lax.fori_loop(0, _NCHUNK, chunk, 0)
    pltpu.sync_copy(acc_v, out.at[pl.ds(wid * _ROWS_PER_W, _ROWS_PER_W)])


_pool_call = functools.partial(
    pl.kernel,
    mesh=plsc.VectorSubcoreMesh(core_axis_name="c", subcore_axis_name="s"),
    out_type=jax.ShapeDtypeStruct((_BATCH, _EMBED), jnp.float32),
    scratch_types=[
        pltpu.VMEM((_NCHUNK, _CHUNK_IDX), jnp.int32),
        pltpu.VMEM((_CHUNK_IDX, _EMBED), jnp.float32),
        pltpu.VMEM((_ROWS_PER_W, _EMBED), jnp.float32),
        pltpu.SemaphoreType.DMA,
    ],
    compiler_params=pltpu.CompilerParams(use_tc_tiling_on_sc=False),
)(_pool_body)


_BN = 1024


def _mlp_body(pooled_ref, W1_ref, b1_ref, W2_ref, b2_ref, out_ref, h_ref):
    @pl.when(pl.program_id(0) == 0)
    def _():
        h = jnp.dot(pooled_ref[...], W1_ref[...], preferred_element_type=jnp.float32)
        h_ref[...] = jnp.maximum(h + b1_ref[...], 0.0).astype(jnp.bfloat16)

    w2 = W2_ref[...].astype(jnp.bfloat16)
    out_ref[...] = (
        jnp.dot(h_ref[...], w2, preferred_element_type=jnp.float32) + b2_ref[...]
    )


_mlp_call = pl.pallas_call(
    _mlp_body,
    grid=(pl.cdiv(_VOCAB, _BN),),
    in_specs=[
        pl.BlockSpec((_BATCH, _EMBED), lambda j: (0, 0)),
        pl.BlockSpec((_EMBED, _HIDDEN), lambda j: (0, 0)),
        pl.BlockSpec((1, _HIDDEN), lambda j: (0, 0)),
        pl.BlockSpec((_HIDDEN, _BN), lambda j: (0, j)),
        pl.BlockSpec((1, _BN), lambda j: (0, j)),
    ],
    out_specs=pl.BlockSpec((_BATCH, _BN), lambda j: (0, j)),
    out_shape=jax.ShapeDtypeStruct((_BATCH, _VOCAB), jnp.float32),
    scratch_shapes=[pltpu.VMEM((_BATCH, _HIDDEN), jnp.bfloat16)],
)


def kernel(src, emb_table, W1, b1, W2, b2):
    src_r = src.reshape(_NW, _NCHUNK, _CHUNK_IDX).astype(jnp.int32)
    pooled = _pool_call(src_r, emb_table)
    return _mlp_call(pooled, W1, b1.reshape(1, -1), W2, b2.reshape(1, -1))


# speedup vs baseline: 1.0022x; 1.0014x over previous
import functools

import jax
import jax.numpy as jnp
from jax import lax
from jax.experimental import pallas as pl
from jax.experimental.pallas import tpu as pltpu
from jax.experimental.pallas import tpu_sc as plsc

_VOCAB = 100000
_EMBED = 64
_HIDDEN = 128
_BATCH = 4096
_SEQ = 50

_NC = 2
_NS = 16
_NW = _NC * _NS
_ROWS_PER_W = _BATCH // _NW
_CHUNK_ROWS = 2
_CHUNK_IDX = _CHUNK_ROWS * _SEQ
_NCHUNK = _ROWS_PER_W // _CHUNK_ROWS


def _pool_body(src_r, table, out, idx_v, rows_v, acc_v, sem):
    wid = lax.axis_index("s") * _NC + lax.axis_index("c")
    pltpu.sync_copy(src_r.at[wid], idx_v)

    def chunk(c, carry):
        pltpu.async_copy(table.at[idx_v.at[c]], rows_v, sem).wait()
        for r in range(_CHUNK_ROWS):
            for k in range(_EMBED // 16):
                acc = rows_v[r * _SEQ, pl.ds(16 * k, 16)]
                for s in range(1, _SEQ):
                    acc = acc + rows_v[r * _SEQ + s, pl.ds(16 * k, 16)]
                acc_v[_CHUNK_ROWS * c + r, pl.ds(16 * k, 16)] = acc * (1.0 / _SEQ)
        return carry

    lax.fori_loop(0, _NCHUNK, chunk, 0)
    pltpu.sync_copy(acc_v, out.at[pl.ds(wid * _ROWS_PER_W, _ROWS_PER_W)])


_pool_call = functools.partial(
    pl.kernel,
    mesh=plsc.VectorSubcoreMesh(core_axis_name="c", subcore_axis_name="s"),
    out_type=jax.ShapeDtypeStruct((_BATCH, _EMBED), jnp.float32),
    scratch_types=[
        pltpu.VMEM((_NCHUNK, _CHUNK_IDX), jnp.int32),
        pltpu.VMEM((_CHUNK_IDX, _EMBED), jnp.float32),
        pltpu.VMEM((_ROWS_PER_W, _EMBED), jnp.float32),
        pltpu.SemaphoreType.DMA,
    ],
    compiler_params=pltpu.CompilerParams(use_tc_tiling_on_sc=False),
)(_pool_body)


_BN = 512
_NSTEPS = pl.cdiv(_VOCAB, _BN)
_TAIL = _VOCAB - (_NSTEPS - 1) * _BN
_RING = 4


def _mlp_body(pooled_ref, W1_ref, b1_ref, W2_ref, b2_ref, out_hbm, h_ref, obuf, tbuf, sems):
    j = pl.program_id(0)
    slot = lax.rem(j, _RING)

    @pl.when(j == 0)
    def _():
        h = jnp.dot(pooled_ref[...], W1_ref[...], preferred_element_type=jnp.float32)
        h_ref[...] = jnp.maximum(h + b1_ref[...], 0.0).astype(jnp.bfloat16)

    @pl.when(j >= _RING)
    def _():
        pltpu.make_async_copy(
            obuf.at[slot],
            out_hbm.at[:, pl.ds((j - _RING) * _BN, _BN)],
            sems.at[slot],
        ).wait()

    w2 = W2_ref[...].astype(jnp.bfloat16)

    @pl.when(j < _NSTEPS - 1)
    def _():
        obuf[slot] = (
            jnp.dot(h_ref[...], w2, preferred_element_type=jnp.float32) + b2_ref[...]
        )
        pltpu.make_async_copy(
            obuf.at[slot],
            out_hbm.at[:, pl.ds(j * _BN, _BN)],
            sems.at[slot],
        ).start()

    @pl.when(j == _NSTEPS - 1)
    def _():
        val = (
            jnp.dot(h_ref[...], w2, preferred_element_type=jnp.float32) + b2_ref[...]
        )
        tbuf[...] = val[:, :_TAIL]
        tail_copy = pltpu.make_async_copy(
            tbuf,
            out_hbm.at[:, pl.ds((_NSTEPS - 1) * _BN, _TAIL)],
            sems.at[(_NSTEPS - 1) % _RING],
        )
        tail_copy.start()
        for jj in range(_NSTEPS - _RING, _NSTEPS - 1):
            pltpu.make_async_copy(
                obuf.at[jj % _RING],
                out_hbm.at[:, pl.ds(jj * _BN, _BN)],
                sems.at[jj % _RING],
            ).wait()
        tail_copy.wait()


_mlp_call = pl.pallas_call(
    _mlp_body,
    grid=(_NSTEPS,),
    in_specs=[
        pl.BlockSpec((_BATCH, _EMBED), lambda j: (0, 0)),
        pl.BlockSpec((_EMBED, _HIDDEN), lambda j: (0, 0)),
        pl.BlockSpec((1, _HIDDEN), lambda j: (0, 0)),
        pl.BlockSpec((_HIDDEN, _BN), lambda j: (0, j)),
        pl.BlockSpec((1, _BN), lambda j: (0, j)),
    ],
    out_specs=pl.BlockSpec(memory_space=pl.ANY),
    out_shape=jax.ShapeDtypeStruct((_BATCH, _VOCAB), jnp.float32),
    scratch_shapes=[
        pltpu.VMEM((_BATCH, _HIDDEN), jnp.bfloat16),
        pltpu.VMEM((_RING, _BATCH, _BN), jnp.float32),
        pltpu.VMEM((_BATCH, _TAIL), jnp.float32),
        pltpu.SemaphoreType.DMA((_RING,)),
    ],
)


def kernel(src, emb_table, W1, b1, W2, b2):
    src_r = src.reshape(_NW, _NCHUNK, _CHUNK_IDX).astype(jnp.int32)
    pooled = _pool_call(src_r, emb_table)
    return _mlp_call(pooled, W1, b1.reshape(1, -1), W2, b2.reshape(1, -1))
